# baseline (speedup 1.0000x reference)
import jax
import jax.numpy as jnp
from jax.experimental import pallas as pl
from jax.experimental.pallas import tpu as pltpu

_T, _H, _W, _D = 32, 64, 64, 128
_NBUF = 6
_CH = _H * _W


def _body(st_ref, sh_ref, sw_ref, t_ref, h_ref, w_ref, o_ref, buf, sem):
    i = pl.program_id(0)
    b = jax.lax.rem(i, _NBUF)

    @pl.when(i >= _NBUF)
    def _drain():
        pltpu.make_async_copy(buf.at[b], o_ref.at[0, pl.ds((i - _NBUF) * _CH, _CH), :], sem.at[b]).wait()

    ts = t_ref[0, 0, :] * st_ref[0]
    hs = h_ref[...] * sh_ref[0]
    ws = w_ref[...] * sw_ref[0]
    th = ts[None, :] + hs
    out = th[:, None, :] + ws[None, :, :]
    buf[b] = out.reshape(_CH, _D)
    pltpu.make_async_copy(buf.at[b], o_ref.at[0, pl.ds(i * _CH, _CH), :], sem.at[b]).start()

    @pl.when(i == _T - 1)
    def _final():
        for k in range(_NBUF):
            j = _T - _NBUF + k
            bb = jax.lax.rem(jnp.int32(j), _NBUF)
            pltpu.make_async_copy(buf.at[bb], o_ref.at[0, pl.ds(j * _CH, _CH), :], sem.at[bb]).wait()


def kernel(t, h, w, temporal_embed, height_embed, width_embed, scale_t, scale_h, scale_w):
    return pl.pallas_call(
        _body,
        grid=(_T,),
        in_specs=[
            pl.BlockSpec(memory_space=pltpu.SMEM),
            pl.BlockSpec(memory_space=pltpu.SMEM),
            pl.BlockSpec(memory_space=pltpu.SMEM),
            pl.BlockSpec((1, 1, _D), lambda i: (i, 0, 0)),
            pl.BlockSpec((_H, _D), lambda i: (0, 0)),
            pl.BlockSpec((_W, _D), lambda i: (0, 0)),
        ],
        out_specs=pl.BlockSpec(memory_space=pl.ANY),
        out_shape=jax.ShapeDtypeStruct((1, _T * _H * _W, _D), jnp.float32),
        scratch_shapes=[
            pltpu.VMEM((_NBUF, _CH, _D), jnp.float32),
            pltpu.SemaphoreType.DMA((_NBUF,)),
        ],
    )(scale_t, scale_h, scale_w,
      temporal_embed.reshape(temporal_embed.shape[0], 1, _D), height_embed, width_embed)


# speedup vs baseline: 1.0507x; 1.0507x over previous
import jax
import jax.numpy as jnp
from jax.experimental import pallas as pl
from jax.experimental.pallas import tpu as pltpu

_T, _H, _W, _D = 32, 64, 64, 128
_NBUF = 3
_CH = _H * _W


def _body(st_ref, sh_ref, sw_ref, t_ref, h_ref, w_ref, o_ref, buf, sem):
    i = pl.program_id(0)
    b = jax.lax.rem(i, _NBUF)

    @pl.when(i >= _NBUF)
    def _drain():
        pltpu.make_async_copy(buf.at[b], o_ref.at[0, pl.ds((i - _NBUF) * _CH, _CH), :], sem.at[b]).wait()

    ts = t_ref[0, 0, :] * st_ref[0]
    hs = h_ref[...] * sh_ref[0]
    ws = w_ref[...] * sw_ref[0]
    th = ts[None, :] + hs
    out = th[:, None, :] + ws[None, :, :]
    buf[b] = out.reshape(_CH, _D)
    pltpu.make_async_copy(buf.at[b], o_ref.at[0, pl.ds(i * _CH, _CH), :], sem.at[b]).start()

    @pl.when(i == _T - 1)
    def _final():
        for k in range(_NBUF):
            j = _T - _NBUF + k
            bb = jax.lax.rem(jnp.int32(j), _NBUF)
            pltpu.make_async_copy(buf.at[bb], o_ref.at[0, pl.ds(j * _CH, _CH), :], sem.at[bb]).wait()


def kernel(t, h, w, temporal_embed, height_embed, width_embed, scale_t, scale_h, scale_w):
    return pl.pallas_call(
        _body,
        grid=(_T,),
        in_specs=[
            pl.BlockSpec(memory_space=pltpu.SMEM),
            pl.BlockSpec(memory_space=pltpu.SMEM),
            pl.BlockSpec(memory_space=pltpu.SMEM),
            pl.BlockSpec((1, 1, _D), lambda i: (i, 0, 0)),
            pl.BlockSpec((_H, _D), lambda i: (0, 0)),
            pl.BlockSpec((_W, _D), lambda i: (0, 0)),
        ],
        out_specs=pl.BlockSpec(memory_space=pl.ANY),
        out_shape=jax.ShapeDtypeStruct((1, _T * _H * _W, _D), jnp.float32),
        scratch_shapes=[
            pltpu.VMEM((_NBUF, _CH, _D), jnp.float32),
            pltpu.SemaphoreType.DMA((_NBUF,)),
        ],
    )(scale_t, scale_h, scale_w,
      temporal_embed.reshape(temporal_embed.shape[0], 1, _D), height_embed, width_embed)
